# baseline (speedup 1.0000x reference)
import functools

import jax
import jax.numpy as jnp
from jax.experimental import pallas as pl
from jax.experimental.pallas import tpu as pltpu

T = 2048
D = 1024
FF = 1024
E = 8
TOP_K = 2
P = T * TOP_K
BM = 512
NB = P // BM + E
NPAD = NB * BM


def _cumsum_rows(a):
    s = 1
    while s < T:
        a = a + jnp.concatenate([jnp.zeros((s, E), a.dtype), a[:-s]], axis=0)
        s *= 2
    return a


def _cumsum_lanes(a):
    s = 1
    while s < E:
        a = a + jnp.concatenate([jnp.zeros((1, s), a.dtype), a[:, :-s]], axis=1)
        s *= 2
    return a


def _router_dispatch_kernel(x_ref, rw_ref, bias_ref, idx_ref, w_ref, ends_ref):
    x = x_ref[...]
    logits = jax.lax.dot_general(
        x, rw_ref[...], (((1,), (1,)), ((), ())),
        preferred_element_type=jnp.float32,
        precision=jax.lax.Precision.DEFAULT)
    m = jnp.max(logits, axis=-1, keepdims=True)
    ex = jnp.exp(logits - m)
    scores = ex / jnp.sum(ex, axis=-1, keepdims=True)
    sel = scores + bias_ref[...]
    lane = jax.lax.broadcasted_iota(jnp.int32, (T, E), 1)
    BIG = jnp.int32(2 * E)
    NEG = jnp.float32(-1e30)
    m1 = jnp.max(sel, axis=-1, keepdims=True)
    i1 = jnp.min(jnp.where(sel == m1, lane, BIG), axis=-1, keepdims=True)
    oh1 = lane == i1
    sel2 = jnp.where(oh1, NEG, sel)
    m2 = jnp.max(sel2, axis=-1, keepdims=True)
    i2 = jnp.min(jnp.where(sel2 == m2, lane, BIG), axis=-1, keepdims=True)
    oh2 = lane == i2
    w1 = jnp.sum(jnp.where(oh1, scores, 0.0), axis=-1, keepdims=True)
    w2 = jnp.sum(jnp.where(oh2, scores, 0.0), axis=-1, keepdims=True)

    ohk = (oh1 | oh2).astype(jnp.int32)
    csum = _cumsum_rows(ohk)
    cexc = csum - ohk
    counts = csum[T - 1:T, :]
    padded = ((counts + BM - 1) // BM) * BM
    ends = _cumsum_lanes(padded)
    start = ends - padded
    slot = start + cexc
    d0 = jnp.sum(jnp.where(oh1, slot, 0), axis=-1, keepdims=True)
    d1 = jnp.sum(jnp.where(oh2, slot, 0), axis=-1, keepdims=True)
    idx_ref[...] = jnp.where(lane == 0, d0, jnp.where(lane == 1, d1, 0))
    w_ref[...] = jnp.where(lane == 0, w1, jnp.where(lane == 1, w2, 0.0))
    ends_ref[...] = ends


def _expert_of(i, ends_ref):
    b = i * BM
    e = jnp.int32(0)
    for k in range(E):
        e = e + jnp.where(b >= ends_ref[k], 1, 0).astype(jnp.int32)
    return e


def _grouped_kernel(ends_ref, xs_ref, wg_ref, wu_ref, wd_ref, ys_ref):
    i = pl.program_id(0)

    @pl.when(i * BM < ends_ref[E - 1])
    def _():
        xb = xs_ref[...].astype(jnp.bfloat16)
        g = jax.lax.dot_general(xb, wg_ref[0].astype(jnp.bfloat16),
                                (((1,), (1,)), ((), ())),
                                preferred_element_type=jnp.float32)
        u = jax.lax.dot_general(xb, wu_ref[0].astype(jnp.bfloat16),
                                (((1,), (1,)), ((), ())),
                                preferred_element_type=jnp.float32)
        h = (g * jax.lax.logistic(g)) * u
        ys_ref[...] = jax.lax.dot_general(
            h.astype(jnp.bfloat16), wd_ref[0].astype(jnp.bfloat16),
            (((1,), (1,)), ((), ())),
            preferred_element_type=jnp.float32)


def kernel(hidden_states, router_w, correction_bias, w_gate, w_up, w_down,
           num_global_tokens, max_num_tokens_per_gpu):
    x = hidden_states
    bias = correction_bias.reshape(1, E).astype(jnp.float32)

    idx, w, ends = pl.pallas_call(
        _router_dispatch_kernel,
        grid=(1,),
        in_specs=[
            pl.BlockSpec((T, D), lambda i: (0, 0)),
            pl.BlockSpec((E, D), lambda i: (0, 0)),
            pl.BlockSpec((1, E), lambda i: (0, 0)),
        ],
        out_specs=[
            pl.BlockSpec((T, E), lambda i: (0, 0)),
            pl.BlockSpec((T, E), lambda i: (0, 0)),
            pl.BlockSpec((1, E), lambda i: (0, 0)),
        ],
        out_shape=[
            jax.ShapeDtypeStruct((T, E), jnp.int32),
            jax.ShapeDtypeStruct((T, E), jnp.float32),
            jax.ShapeDtypeStruct((1, E), jnp.int32),
        ],
    )(x, router_w, bias)

    d0 = idx[:, 0]
    d1 = idx[:, 1]
    tok = jax.lax.iota(jnp.int32, T)
    slot_token = (jnp.zeros((NPAD,), jnp.int32).at[d0].set(tok)
                  .at[d1].set(tok))
    xs = jnp.take(x, slot_token, axis=0)

    ys = pl.pallas_call(
        _grouped_kernel,
        grid_spec=pltpu.PrefetchScalarGridSpec(
            num_scalar_prefetch=1,
            grid=(NB,),
            in_specs=[
                pl.BlockSpec((BM, D), lambda i, ends: (i, 0)),
                pl.BlockSpec((1, FF, D),
                             lambda i, ends: (jnp.minimum(_expert_of(i, ends),
                                                          E - 1), 0, 0)),
                pl.BlockSpec((1, FF, D),
                             lambda i, ends: (jnp.minimum(_expert_of(i, ends),
                                                          E - 1), 0, 0)),
                pl.BlockSpec((1, D, FF),
                             lambda i, ends: (jnp.minimum(_expert_of(i, ends),
                                                          E - 1), 0, 0)),
            ],
            out_specs=pl.BlockSpec((BM, D), lambda i, ends: (i, 0)),
        ),
        out_shape=jax.ShapeDtypeStruct((NPAD, D), jnp.float32),
    )(ends.reshape(E), xs, w_gate, w_up, w_down)

    return ys, idx, w
    out = (w[:, 0:1] * jnp.take(ys, d0, axis=0)
           + w[:, 1:2] * jnp.take(ys, d1, axis=0))
    return out


# speedup vs baseline: 1.1165x; 1.1165x over previous
import functools

import jax
import jax.numpy as jnp
from jax import lax
from jax.experimental import pallas as pl
from jax.experimental.pallas import tpu as pltpu
from jax.experimental.pallas import tpu_sc as plsc

T = 2048
D = 1024
FF = 1024
E = 8
TOP_K = 2
P = T * TOP_K
BM = 256
NB = P // BM + E
NPAD = NB * BM


def _cumsum_rows(a):
    s = 1
    while s < T:
        a = a + jnp.concatenate([jnp.zeros((s, E), a.dtype), a[:-s]], axis=0)
        s *= 2
    return a


def _cumsum_lanes(a):
    s = 1
    while s < E:
        a = a + jnp.concatenate([jnp.zeros((1, s), a.dtype), a[:, :-s]], axis=1)
        s *= 2
    return a


def _router_dispatch_kernel(x_ref, rw_ref, bias_ref, idx_ref, w_ref, ends_ref):
    x = x_ref[...]
    logits = jax.lax.dot_general(
        x, rw_ref[...], (((1,), (1,)), ((), ())),
        preferred_element_type=jnp.float32,
        precision=jax.lax.Precision.DEFAULT)
    m = jnp.max(logits, axis=-1, keepdims=True)
    ex = jnp.exp(logits - m)
    scores = ex / jnp.sum(ex, axis=-1, keepdims=True)
    sel = scores + bias_ref[...]
    lane = jax.lax.broadcasted_iota(jnp.int32, (T, E), 1)
    BIG = jnp.int32(2 * E)
    NEG = jnp.float32(-1e30)
    m1 = jnp.max(sel, axis=-1, keepdims=True)
    i1 = jnp.min(jnp.where(sel == m1, lane, BIG), axis=-1, keepdims=True)
    oh1 = lane == i1
    sel2 = jnp.where(oh1, NEG, sel)
    m2 = jnp.max(sel2, axis=-1, keepdims=True)
    i2 = jnp.min(jnp.where(sel2 == m2, lane, BIG), axis=-1, keepdims=True)
    oh2 = lane == i2
    w1 = jnp.sum(jnp.where(oh1, scores, 0.0), axis=-1, keepdims=True)
    w2 = jnp.sum(jnp.where(oh2, scores, 0.0), axis=-1, keepdims=True)

    ohk = (oh1 | oh2).astype(jnp.int32)
    csum = _cumsum_rows(ohk)
    cexc = csum - ohk
    counts = csum[T - 1:T, :]
    padded = ((counts + BM - 1) // BM) * BM
    ends = _cumsum_lanes(padded)
    start = ends - padded
    slot = start + cexc
    d0 = jnp.sum(jnp.where(oh1, slot, 0), axis=-1, keepdims=True)
    d1 = jnp.sum(jnp.where(oh2, slot, 0), axis=-1, keepdims=True)
    idx_ref[...] = jnp.where(lane == 0, d0, jnp.where(lane == 1, d1, 0))
    w_ref[...] = jnp.where(lane == 0, w1, jnp.where(lane == 1, w2, 0.0))
    ends_ref[...] = ends


_NW = 32
_CHUNK = P // _NW
_RB = 64


def _dispatch_sc_kernel(x_hbm, tok_hbm, d_hbm, xs_hbm, tok_v, d_v, rows_v,
                        sem):
    wid = lax.axis_index("s") * 2 + lax.axis_index("c")
    base = wid * _CHUNK
    for c in range(_CHUNK // _RB):
        off = base + c * _RB
        pltpu.sync_copy(tok_hbm.at[pl.ds(off, _RB)], tok_v)
        pltpu.sync_copy(d_hbm.at[pl.ds(off, _RB)], d_v)
        pltpu.async_copy(x_hbm.at[tok_v], rows_v, sem).wait()
        pltpu.async_copy(rows_v, xs_hbm.at[d_v], sem).wait()


def _expert_of(i, ends_ref):
    b = i * BM
    e = jnp.int32(0)
    for k in range(E):
        e = e + jnp.where(b >= ends_ref[k], 1, 0).astype(jnp.int32)
    return e


def _grouped_kernel(ends_ref, xs_ref, wg_ref, wu_ref, wd_ref, ys_ref):
    i = pl.program_id(0)

    @pl.when(i * BM < ends_ref[E - 1])
    def _():
        xb = xs_ref[...].astype(jnp.bfloat16)
        g = jax.lax.dot_general(xb, wg_ref[0].astype(jnp.bfloat16),
                                (((1,), (1,)), ((), ())),
                                preferred_element_type=jnp.float32)
        u = jax.lax.dot_general(xb, wu_ref[0].astype(jnp.bfloat16),
                                (((1,), (1,)), ((), ())),
                                preferred_element_type=jnp.float32)
        h = (g * jax.lax.logistic(g)) * u
        ys_ref[...] = jax.lax.dot_general(
            h.astype(jnp.bfloat16), wd_ref[0].astype(jnp.bfloat16),
            (((1,), (1,)), ((), ())),
            preferred_element_type=jnp.float32)


def kernel(hidden_states, router_w, correction_bias, w_gate, w_up, w_down,
           num_global_tokens, max_num_tokens_per_gpu):
    x = hidden_states
    bias = correction_bias.reshape(1, E).astype(jnp.float32)

    idx, w, ends = pl.pallas_call(
        _router_dispatch_kernel,
        grid=(1,),
        in_specs=[
            pl.BlockSpec((T, D), lambda i: (0, 0)),
            pl.BlockSpec((E, D), lambda i: (0, 0)),
            pl.BlockSpec((1, E), lambda i: (0, 0)),
        ],
        out_specs=[
            pl.BlockSpec((T, E), lambda i: (0, 0)),
            pl.BlockSpec((T, E), lambda i: (0, 0)),
            pl.BlockSpec((1, E), lambda i: (0, 0)),
        ],
        out_shape=[
            jax.ShapeDtypeStruct((T, E), jnp.int32),
            jax.ShapeDtypeStruct((T, E), jnp.float32),
            jax.ShapeDtypeStruct((1, E), jnp.int32),
        ],
    )(x, router_w, bias)

    d0 = idx[:, 0]
    d1 = idx[:, 1]
    d_all = idx[:, :TOP_K].reshape(P)
    tok_all = jnp.repeat(jax.lax.iota(jnp.int32, T), TOP_K)

    xs = pl.kernel(
        _dispatch_sc_kernel,
        out_type=jax.ShapeDtypeStruct((NPAD, D), jnp.float32),
        mesh=plsc.VectorSubcoreMesh(core_axis_name="c", subcore_axis_name="s"),
        scratch_types=[
            pltpu.VMEM((_RB,), jnp.int32),
            pltpu.VMEM((_RB,), jnp.int32),
            pltpu.VMEM((_RB, D), jnp.float32),
            pltpu.SemaphoreType.DMA,
        ],
    )(x, tok_all, d_all)

    ys = pl.pallas_call(
        _grouped_kernel,
        grid_spec=pltpu.PrefetchScalarGridSpec(
            num_scalar_prefetch=1,
            grid=(NB,),
            in_specs=[
                pl.BlockSpec((BM, D), lambda i, ends: (i, 0)),
                pl.BlockSpec((1, FF, D),
                             lambda i, ends: (jnp.minimum(_expert_of(i, ends),
                                                          E - 1), 0, 0)),
                pl.BlockSpec((1, FF, D),
                             lambda i, ends: (jnp.minimum(_expert_of(i, ends),
                                                          E - 1), 0, 0)),
                pl.BlockSpec((1, D, FF),
                             lambda i, ends: (jnp.minimum(_expert_of(i, ends),
                                                          E - 1), 0, 0)),
            ],
            out_specs=pl.BlockSpec((BM, D), lambda i, ends: (i, 0)),
        ),
        out_shape=jax.ShapeDtypeStruct((NPAD, D), jnp.float32),
    )(ends.reshape(E), xs, w_gate, w_up, w_down)

    out = (w[:, 0:1] * jnp.take(ys, d0, axis=0)
           + w[:, 1:2] * jnp.take(ys, d1, axis=0))
    return out


# speedup vs baseline: 1.1255x; 1.0081x over previous
import functools

import jax
import jax.numpy as jnp
from jax import lax
from jax.experimental import pallas as pl
from jax.experimental.pallas import tpu as pltpu
from jax.experimental.pallas import tpu_sc as plsc

T = 2048
D = 1024
FF = 1024
E = 8
TOP_K = 2
P = T * TOP_K
BM = 256
NB = P // BM + E
NPAD = NB * BM


def _cumsum_rows(a):
    s = 1
    while s < T:
        a = a + jnp.concatenate([jnp.zeros((s, E), a.dtype), a[:-s]], axis=0)
        s *= 2
    return a


def _cumsum_lanes(a):
    s = 1
    while s < E:
        a = a + jnp.concatenate([jnp.zeros((1, s), a.dtype), a[:, :-s]], axis=1)
        s *= 2
    return a


def _router_dispatch_kernel(x_ref, rw_ref, bias_ref, idx_ref, w_ref, ends_ref):
    x = x_ref[...]
    logits = jax.lax.dot_general(
        x, rw_ref[...], (((1,), (1,)), ((), ())),
        preferred_element_type=jnp.float32,
        precision=jax.lax.Precision.DEFAULT)
    m = jnp.max(logits, axis=-1, keepdims=True)
    ex = jnp.exp(logits - m)
    scores = ex / jnp.sum(ex, axis=-1, keepdims=True)
    sel = scores + bias_ref[...]
    lane = jax.lax.broadcasted_iota(jnp.int32, (T, E), 1)
    BIG = jnp.int32(2 * E)
    NEG = jnp.float32(-1e30)
    m1 = jnp.max(sel, axis=-1, keepdims=True)
    i1 = jnp.min(jnp.where(sel == m1, lane, BIG), axis=-1, keepdims=True)
    oh1 = lane == i1
    sel2 = jnp.where(oh1, NEG, sel)
    m2 = jnp.max(sel2, axis=-1, keepdims=True)
    i2 = jnp.min(jnp.where(sel2 == m2, lane, BIG), axis=-1, keepdims=True)
    oh2 = lane == i2
    w1 = jnp.sum(jnp.where(oh1, scores, 0.0), axis=-1, keepdims=True)
    w2 = jnp.sum(jnp.where(oh2, scores, 0.0), axis=-1, keepdims=True)

    ohk = (oh1 | oh2).astype(jnp.int32)
    csum = _cumsum_rows(ohk)
    cexc = csum - ohk
    counts = csum[T - 1:T, :]
    padded = ((counts + BM - 1) // BM) * BM
    ends = _cumsum_lanes(padded)
    start = ends - padded
    slot = start + cexc
    d0 = jnp.sum(jnp.where(oh1, slot, 0), axis=-1, keepdims=True)
    d1 = jnp.sum(jnp.where(oh2, slot, 0), axis=-1, keepdims=True)
    idx_ref[...] = jnp.where(lane == 0, d0, jnp.where(lane == 1, d1, 0))
    w_ref[...] = jnp.where(lane == 0, w1, jnp.where(lane == 1, w2, 0.0))
    ends_ref[...] = ends


_NW = 32
_CHUNK = P // _NW
_RB = 64


def _dispatch_sc_kernel(x_hbm, tok_hbm, d_hbm, xs_hbm, tok_v, d_v, rows_v,
                        sem):
    wid = lax.axis_index("s") * 2 + lax.axis_index("c")
    base = wid * _CHUNK
    for c in range(_CHUNK // _RB):
        off = base + c * _RB
        pltpu.sync_copy(tok_hbm.at[pl.ds(off, _RB)], tok_v)
        pltpu.sync_copy(d_hbm.at[pl.ds(off, _RB)], d_v)
        pltpu.async_copy(x_hbm.at[tok_v], rows_v, sem).wait()
        pltpu.async_copy(rows_v, xs_hbm.at[d_v], sem).wait()


_RC = 32
_TPW = T // _NW


def _combine_sc_kernel(ys_hbm, d0_hbm, d1_hbm, w0_hbm, w1_hbm, out_hbm,
                       d0_v, d1_v, w0_v, w1_v, a_v, b_v, sem):
    wid = lax.axis_index("s") * 2 + lax.axis_index("c")
    base = wid * _TPW
    for c in range(_TPW // _RC):
        toff = base + c * _RC
        pltpu.sync_copy(d0_hbm.at[pl.ds(toff, _RC)], d0_v)
        pltpu.sync_copy(d1_hbm.at[pl.ds(toff, _RC)], d1_v)
        pltpu.sync_copy(w0_hbm.at[pl.ds(toff, _RC)], w0_v)
        pltpu.sync_copy(w1_hbm.at[pl.ds(toff, _RC)], w1_v)
        pltpu.async_copy(ys_hbm.at[d0_v], a_v, sem).wait()
        pltpu.async_copy(ys_hbm.at[d1_v], b_v, sem).wait()

        def _row(j, _):
            w0 = w0_v[j]
            w1 = w1_v[j]
            for v in range(D // 16):
                sl = pl.ds(v * 16, 16)
                a_v[j, sl] = a_v[j, sl] * w0 + b_v[j, sl] * w1
            return 0

        lax.fori_loop(0, _RC, _row, 0)
        pltpu.sync_copy(a_v, out_hbm.at[pl.ds(toff, _RC)])


def _expert_of(i, ends_ref):
    b = i * BM
    e = jnp.int32(0)
    for k in range(E):
        e = e + jnp.where(b >= ends_ref[k], 1, 0).astype(jnp.int32)
    return e


def _grouped_kernel(ends_ref, xs_ref, wg_ref, wu_ref, wd_ref, ys_ref):
    i = pl.program_id(0)

    @pl.when(i * BM < ends_ref[E - 1])
    def _():
        xb = xs_ref[...].astype(jnp.bfloat16)
        g = jax.lax.dot_general(xb, wg_ref[0].astype(jnp.bfloat16),
                                (((1,), (1,)), ((), ())),
                                preferred_element_type=jnp.float32)
        u = jax.lax.dot_general(xb, wu_ref[0].astype(jnp.bfloat16),
                                (((1,), (1,)), ((), ())),
                                preferred_element_type=jnp.float32)
        h = (g * jax.lax.logistic(g)) * u
        ys_ref[...] = jax.lax.dot_general(
            h.astype(jnp.bfloat16), wd_ref[0].astype(jnp.bfloat16),
            (((1,), (1,)), ((), ())),
            preferred_element_type=jnp.float32)


def kernel(hidden_states, router_w, correction_bias, w_gate, w_up, w_down,
           num_global_tokens, max_num_tokens_per_gpu):
    x = hidden_states
    bias = correction_bias.reshape(1, E).astype(jnp.float32)

    idx, w, ends = pl.pallas_call(
        _router_dispatch_kernel,
        grid=(1,),
        in_specs=[
            pl.BlockSpec((T, D), lambda i: (0, 0)),
            pl.BlockSpec((E, D), lambda i: (0, 0)),
            pl.BlockSpec((1, E), lambda i: (0, 0)),
        ],
        out_specs=[
            pl.BlockSpec((T, E), lambda i: (0, 0)),
            pl.BlockSpec((T, E), lambda i: (0, 0)),
            pl.BlockSpec((1, E), lambda i: (0, 0)),
        ],
        out_shape=[
            jax.ShapeDtypeStruct((T, E), jnp.int32),
            jax.ShapeDtypeStruct((T, E), jnp.float32),
            jax.ShapeDtypeStruct((1, E), jnp.int32),
        ],
    )(x, router_w, bias)

    d0 = idx[:, 0]
    d1 = idx[:, 1]
    d_all = idx[:, :TOP_K].reshape(P)
    tok_all = jnp.repeat(jax.lax.iota(jnp.int32, T), TOP_K)

    xs = pl.kernel(
        _dispatch_sc_kernel,
        out_type=jax.ShapeDtypeStruct((NPAD, D), jnp.float32),
        mesh=plsc.VectorSubcoreMesh(core_axis_name="c", subcore_axis_name="s"),
        scratch_types=[
            pltpu.VMEM((_RB,), jnp.int32),
            pltpu.VMEM((_RB,), jnp.int32),
            pltpu.VMEM((_RB, D), jnp.float32),
            pltpu.SemaphoreType.DMA,
        ],
    )(x, tok_all, d_all)

    ys = pl.pallas_call(
        _grouped_kernel,
        grid_spec=pltpu.PrefetchScalarGridSpec(
            num_scalar_prefetch=1,
            grid=(NB,),
            in_specs=[
                pl.BlockSpec((BM, D), lambda i, ends: (i, 0)),
                pl.BlockSpec((1, FF, D),
                             lambda i, ends: (jnp.minimum(_expert_of(i, ends),
                                                          E - 1), 0, 0)),
                pl.BlockSpec((1, FF, D),
                             lambda i, ends: (jnp.minimum(_expert_of(i, ends),
                                                          E - 1), 0, 0)),
                pl.BlockSpec((1, D, FF),
                             lambda i, ends: (jnp.minimum(_expert_of(i, ends),
                                                          E - 1), 0, 0)),
            ],
            out_specs=pl.BlockSpec((BM, D), lambda i, ends: (i, 0)),
        ),
        out_shape=jax.ShapeDtypeStruct((NPAD, D), jnp.float32),
    )(ends.reshape(E), xs, w_gate, w_up, w_down)

    w0w = jnp.broadcast_to(w[:, 0:1], (T, 16))
    w1w = jnp.broadcast_to(w[:, 1:2], (T, 16))
    out = pl.kernel(
        _combine_sc_kernel,
        out_type=jax.ShapeDtypeStruct((T, D), jnp.float32),
        mesh=plsc.VectorSubcoreMesh(core_axis_name="c", subcore_axis_name="s"),
        scratch_types=[
            pltpu.VMEM((_RC,), jnp.int32),
            pltpu.VMEM((_RC,), jnp.int32),
            pltpu.VMEM((_RC, 16), jnp.float32),
            pltpu.VMEM((_RC, 16), jnp.float32),
            pltpu.VMEM((_RC, D), jnp.float32),
            pltpu.VMEM((_RC, D), jnp.float32),
            pltpu.SemaphoreType.DMA,
        ],
    )(ys, d0, d1, w0w, w1w)
    return out


# speedup vs baseline: 1.1393x; 1.0122x over previous
import functools

import jax
import jax.numpy as jnp
from jax import lax
from jax.experimental import pallas as pl
from jax.experimental.pallas import tpu as pltpu
from jax.experimental.pallas import tpu_sc as plsc

T = 2048
D = 1024
FF = 1024
E = 8
TOP_K = 2
P = T * TOP_K
BM = 256
NB = P // BM + E
NPAD = NB * BM


def _cumsum_rows(a):
    s = 1
    while s < T:
        a = a + jnp.concatenate([jnp.zeros((s, E), a.dtype), a[:-s]], axis=0)
        s *= 2
    return a


def _cumsum_lanes(a):
    s = 1
    while s < E:
        a = a + jnp.concatenate([jnp.zeros((1, s), a.dtype), a[:, :-s]], axis=1)
        s *= 2
    return a


def _router_dispatch_kernel(x_ref, rw_ref, bias_ref, idx_ref, w_ref, ends_ref):
    x = x_ref[...]
    logits = jax.lax.dot_general(
        x, rw_ref[...], (((1,), (1,)), ((), ())),
        preferred_element_type=jnp.float32,
        precision=jax.lax.Precision.DEFAULT)
    m = jnp.max(logits, axis=-1, keepdims=True)
    ex = jnp.exp(logits - m)
    scores = ex / jnp.sum(ex, axis=-1, keepdims=True)
    sel = scores + bias_ref[...]
    lane = jax.lax.broadcasted_iota(jnp.int32, (T, E), 1)
    BIG = jnp.int32(2 * E)
    NEG = jnp.float32(-1e30)
    m1 = jnp.max(sel, axis=-1, keepdims=True)
    i1 = jnp.min(jnp.where(sel == m1, lane, BIG), axis=-1, keepdims=True)
    oh1 = lane == i1
    sel2 = jnp.where(oh1, NEG, sel)
    m2 = jnp.max(sel2, axis=-1, keepdims=True)
    i2 = jnp.min(jnp.where(sel2 == m2, lane, BIG), axis=-1, keepdims=True)
    oh2 = lane == i2
    w1 = jnp.sum(jnp.where(oh1, scores, 0.0), axis=-1, keepdims=True)
    w2 = jnp.sum(jnp.where(oh2, scores, 0.0), axis=-1, keepdims=True)

    ohk = (oh1 | oh2).astype(jnp.int32)
    csum = _cumsum_rows(ohk)
    cexc = csum - ohk
    counts = csum[T - 1:T, :]
    padded = ((counts + BM - 1) // BM) * BM
    ends = _cumsum_lanes(padded)
    start = ends - padded
    slot = start + cexc
    d0 = jnp.sum(jnp.where(oh1, slot, 0), axis=-1, keepdims=True)
    d1 = jnp.sum(jnp.where(oh2, slot, 0), axis=-1, keepdims=True)
    idx_ref[...] = jnp.where(lane == 0, d0, jnp.where(lane == 1, d1, 0))
    w_ref[...] = jnp.where(lane == 0, w1, jnp.where(lane == 1, w2, 0.0))
    ends_ref[...] = ends


_NW = 32
_CHUNK = P // _NW
_RB = 64


def _dispatch_sc_kernel(x_hbm, tok_hbm, d_hbm, xs_hbm, tok_v, d_v, rows_v,
                        sem):
    wid = lax.axis_index("s") * 2 + lax.axis_index("c")
    base = wid * _CHUNK
    for c in range(_CHUNK // _RB):
        off = base + c * _RB
        pltpu.sync_copy(tok_hbm.at[pl.ds(off, _RB)], tok_v)
        pltpu.sync_copy(d_hbm.at[pl.ds(off, _RB)], d_v)
        pltpu.async_copy(x_hbm.at[tok_v], rows_v, sem).wait()
        pltpu.async_copy(rows_v, xs_hbm.at[d_v], sem).wait()


_RC = 32
_TPW = T // _NW


def _combine_sc_kernel(ys_hbm, d0_hbm, d1_hbm, w0_hbm, w1_hbm, out_hbm,
                       d0_v, d1_v, w0_v, w1_v, a_v, b_v, sem):
    wid = lax.axis_index("s") * 2 + lax.axis_index("c")
    base = wid * _TPW
    for c in range(_TPW // _RC):
        toff = base + c * _RC
        pltpu.sync_copy(d0_hbm.at[pl.ds(toff, _RC)], d0_v)
        pltpu.sync_copy(d1_hbm.at[pl.ds(toff, _RC)], d1_v)
        pltpu.sync_copy(w0_hbm.at[pl.ds(toff, _RC)], w0_v)
        pltpu.sync_copy(w1_hbm.at[pl.ds(toff, _RC)], w1_v)
        ca = pltpu.async_copy(ys_hbm.at[d0_v], a_v, sem)
        cb = pltpu.async_copy(ys_hbm.at[d1_v], b_v, sem)
        ca.wait()
        cb.wait()

        def _row(j, _):
            w0 = w0_v[j]
            w1 = w1_v[j]
            for v in range(D // 16):
                sl = pl.ds(v * 16, 16)
                a_v[j, sl] = a_v[j, sl] * w0 + b_v[j, sl] * w1
            return 0

        lax.fori_loop(0, _RC, _row, 0)
        pltpu.sync_copy(a_v, out_hbm.at[pl.ds(toff, _RC)])


def _expert_of(i, ends_ref):
    b = i * BM
    e = jnp.int32(0)
    for k in range(E):
        e = e + jnp.where(b >= ends_ref[k], 1, 0).astype(jnp.int32)
    return e


def _grouped_kernel(ends_ref, xs_ref, wg_ref, wu_ref, wd_ref, ys_ref):
    i = pl.program_id(0)

    @pl.when(i * BM < ends_ref[E - 1])
    def _():
        xb = xs_ref[...].astype(jnp.bfloat16)
        g = jax.lax.dot_general(xb, wg_ref[0].astype(jnp.bfloat16),
                                (((1,), (1,)), ((), ())),
                                preferred_element_type=jnp.float32)
        u = jax.lax.dot_general(xb, wu_ref[0].astype(jnp.bfloat16),
                                (((1,), (1,)), ((), ())),
                                preferred_element_type=jnp.float32)
        h = (g * jax.lax.logistic(g)) * u
        ys_ref[...] = jax.lax.dot_general(
            h.astype(jnp.bfloat16), wd_ref[0].astype(jnp.bfloat16),
            (((1,), (1,)), ((), ())),
            preferred_element_type=jnp.float32)


def kernel(hidden_states, router_w, correction_bias, w_gate, w_up, w_down,
           num_global_tokens, max_num_tokens_per_gpu):
    x = hidden_states
    bias = correction_bias.reshape(1, E).astype(jnp.float32)

    idx, w, ends = pl.pallas_call(
        _router_dispatch_kernel,
        grid=(1,),
        in_specs=[
            pl.BlockSpec((T, D), lambda i: (0, 0)),
            pl.BlockSpec((E, D), lambda i: (0, 0)),
            pl.BlockSpec((1, E), lambda i: (0, 0)),
        ],
        out_specs=[
            pl.BlockSpec((T, E), lambda i: (0, 0)),
            pl.BlockSpec((T, E), lambda i: (0, 0)),
            pl.BlockSpec((1, E), lambda i: (0, 0)),
        ],
        out_shape=[
            jax.ShapeDtypeStruct((T, E), jnp.int32),
            jax.ShapeDtypeStruct((T, E), jnp.float32),
            jax.ShapeDtypeStruct((1, E), jnp.int32),
        ],
    )(x, router_w, bias)

    d0 = idx[:, 0]
    d1 = idx[:, 1]
    d_all = idx[:, :TOP_K].reshape(P)
    tok_all = jnp.repeat(jax.lax.iota(jnp.int32, T), TOP_K)

    xs = pl.kernel(
        _dispatch_sc_kernel,
        out_type=jax.ShapeDtypeStruct((NPAD, D), jnp.float32),
        mesh=plsc.VectorSubcoreMesh(core_axis_name="c", subcore_axis_name="s"),
        scratch_types=[
            pltpu.VMEM((_RB,), jnp.int32),
            pltpu.VMEM((_RB,), jnp.int32),
            pltpu.VMEM((_RB, D), jnp.float32),
            pltpu.SemaphoreType.DMA,
        ],
    )(x, tok_all, d_all)

    ys = pl.pallas_call(
        _grouped_kernel,
        grid_spec=pltpu.PrefetchScalarGridSpec(
            num_scalar_prefetch=1,
            grid=(NB,),
            in_specs=[
                pl.BlockSpec((BM, D), lambda i, ends: (i, 0)),
                pl.BlockSpec((1, FF, D),
                             lambda i, ends: (jnp.minimum(_expert_of(i, ends),
                                                          E - 1), 0, 0)),
                pl.BlockSpec((1, FF, D),
                             lambda i, ends: (jnp.minimum(_expert_of(i, ends),
                                                          E - 1), 0, 0)),
                pl.BlockSpec((1, D, FF),
                             lambda i, ends: (jnp.minimum(_expert_of(i, ends),
                                                          E - 1), 0, 0)),
            ],
            out_specs=pl.BlockSpec((BM, D), lambda i, ends: (i, 0)),
        ),
        out_shape=jax.ShapeDtypeStruct((NPAD, D), jnp.float32),
    )(ends.reshape(E), xs, w_gate, w_up, w_down)

    w0w = jnp.broadcast_to(w[:, 0:1], (T, 16))
    w1w = jnp.broadcast_to(w[:, 1:2], (T, 16))
    out = pl.kernel(
        _combine_sc_kernel,
        out_type=jax.ShapeDtypeStruct((T, D), jnp.float32),
        mesh=plsc.VectorSubcoreMesh(core_axis_name="c", subcore_axis_name="s"),
        scratch_types=[
            pltpu.VMEM((_RC,), jnp.int32),
            pltpu.VMEM((_RC,), jnp.int32),
            pltpu.VMEM((_RC, 16), jnp.float32),
            pltpu.VMEM((_RC, 16), jnp.float32),
            pltpu.VMEM((_RC, D), jnp.float32),
            pltpu.VMEM((_RC, D), jnp.float32),
            pltpu.SemaphoreType.DMA,
        ],
    )(ys, d0, d1, w0w, w1w)
    return out
